# initial kernel scaffold (unmeasured)
import jax
import jax.numpy as jnp
from jax import lax
from jax.experimental import pallas as pl
from jax.experimental.pallas import tpu as pltpu

N_DEV = 4


def kernel(Q, K, V):
    b, s, h, d = Q.shape
    scale = d ** -0.5

    def body(q_ref, k_ref, v_ref, out_ref, kbuf, vbuf,
             ksend, krecv, vsend, vrecv):
        my_x = lax.axis_index("x")
        my_y = lax.axis_index("y")
        my_z = lax.axis_index("z")
        left = (my_z - 1) % N_DEV
        right = (my_z + 1) % N_DEV

        barrier = pltpu.get_barrier_semaphore()
        for nbr in (left, right):
            pl.semaphore_signal(
                barrier, inc=1,
                device_id=(my_x, my_y, nbr),
                device_id_type=pl.DeviceIdType.MESH,
            )
        pl.semaphore_wait(barrier, 2)

        kbuf[0] = k_ref[...]
        vbuf[0] = v_ref[...]

        for hop in range(N_DEV - 1):
            k_rdma = pltpu.make_async_remote_copy(
                src_ref=kbuf.at[hop],
                dst_ref=kbuf.at[hop + 1],
                send_sem=ksend.at[hop],
                recv_sem=krecv.at[hop],
                device_id=(my_x, my_y, right),
                device_id_type=pl.DeviceIdType.MESH,
            )
            v_rdma = pltpu.make_async_remote_copy(
                src_ref=vbuf.at[hop],
                dst_ref=vbuf.at[hop + 1],
                send_sem=vsend.at[hop],
                recv_sem=vrecv.at[hop],
                device_id=(my_x, my_y, right),
                device_id_type=pl.DeviceIdType.MESH,
            )
            k_rdma.start()
            v_rdma.start()
            k_rdma.wait()
            v_rdma.wait()

        for bb in range(b):
            for hh in range(h):
                q = q_ref[bb, :, hh, :]
                kk = kbuf[:, bb, :, hh, :].reshape(N_DEV * s, d)
                vv = vbuf[:, bb, :, hh, :].reshape(N_DEV * s, d)
                sc = lax.dot_general(
                    q, kk, (((1,), (1,)), ((), ())),
                    preferred_element_type=jnp.float32,
                ) * scale
                m = jnp.max(sc, axis=1, keepdims=True)
                p = jnp.exp(sc - m)
                l = jnp.sum(p, axis=1, keepdims=True)
                o = lax.dot_general(
                    p, vv, (((1,), (0,)), ((), ())),
                    preferred_element_type=jnp.float32,
                )
                out_ref[bb, :, hh, :] = o / l

    return pl.pallas_call(
        body,
        out_shape=jax.ShapeDtypeStruct((b, s, h, d), jnp.float32),
        in_specs=[
            pl.BlockSpec(memory_space=pltpu.VMEM),
            pl.BlockSpec(memory_space=pltpu.VMEM),
            pl.BlockSpec(memory_space=pltpu.VMEM),
        ],
        out_specs=pl.BlockSpec(memory_space=pltpu.VMEM),
        scratch_shapes=[
            pltpu.VMEM((N_DEV, b, s, h, d), jnp.float32),
            pltpu.VMEM((N_DEV, b, s, h, d), jnp.float32),
            pltpu.SemaphoreType.DMA((N_DEV - 1,)),
            pltpu.SemaphoreType.DMA((N_DEV - 1,)),
            pltpu.SemaphoreType.DMA((N_DEV - 1,)),
            pltpu.SemaphoreType.DMA((N_DEV - 1,)),
        ],
        compiler_params=pltpu.CompilerParams(collective_id=0),
    )(Q, K, V)


# baseline (device time: 186691 ns/iter reference)
import jax
import jax.numpy as jnp
from jax import lax
from jax.experimental import pallas as pl
from jax.experimental.pallas import tpu as pltpu

N_DEV = 4


def kernel(Q, K, V):
    b, s, h, d = Q.shape
    scale = d ** -0.5

    def body(qT_ref, kT_ref, vT_ref, oT_ref, kbuf, vbuf,
             ksend, krecv, vsend, vrecv):
        my_x = lax.axis_index("x")
        my_y = lax.axis_index("y")
        my_z = lax.axis_index("z")
        left = (my_z - 1) % N_DEV
        right = (my_z + 1) % N_DEV

        barrier = pltpu.get_barrier_semaphore()
        for nbr in (left, right):
            pl.semaphore_signal(
                barrier, inc=1,
                device_id=(my_x, my_y, nbr),
                device_id_type=pl.DeviceIdType.MESH,
            )
        pl.semaphore_wait(barrier, 2)

        kbuf[0] = kT_ref[...]
        vbuf[0] = vT_ref[...]

        for hop in range(N_DEV - 1):
            k_rdma = pltpu.make_async_remote_copy(
                src_ref=kbuf.at[hop],
                dst_ref=kbuf.at[hop + 1],
                send_sem=ksend.at[hop],
                recv_sem=krecv.at[hop],
                device_id=(my_x, my_y, right),
                device_id_type=pl.DeviceIdType.MESH,
            )
            v_rdma = pltpu.make_async_remote_copy(
                src_ref=vbuf.at[hop],
                dst_ref=vbuf.at[hop + 1],
                send_sem=vsend.at[hop],
                recv_sem=vrecv.at[hop],
                device_id=(my_x, my_y, right),
                device_id_type=pl.DeviceIdType.MESH,
            )
            k_rdma.start()
            v_rdma.start()
            k_rdma.wait()
            v_rdma.wait()

        def loop_body(i, carry):
            bb = i // h
            hh = i % h
            qT = qT_ref[bb, hh]
            kT = jnp.concatenate(
                [kbuf[sl, bb, hh] for sl in range(N_DEV)], axis=-1
            )
            vT = jnp.concatenate(
                [vbuf[sl, bb, hh] for sl in range(N_DEV)], axis=-1
            )
            sc = lax.dot_general(
                qT, kT, (((0,), (0,)), ((), ())),
                preferred_element_type=jnp.float32,
            ) * scale
            m = jnp.max(sc, axis=1, keepdims=True)
            p = jnp.exp(sc - m)
            p = p / jnp.sum(p, axis=1, keepdims=True)
            oT = lax.dot_general(
                vT, p, (((1,), (1,)), ((), ())),
                preferred_element_type=jnp.float32,
            )
            oT_ref[bb, hh] = oT
            return carry

        lax.fori_loop(0, b * h, loop_body, 0)

    qT = jnp.transpose(Q, (0, 2, 3, 1))
    kT = jnp.transpose(K, (0, 2, 3, 1))
    vT = jnp.transpose(V, (0, 2, 3, 1))

    oT = pl.pallas_call(
        body,
        out_shape=jax.ShapeDtypeStruct((b, h, d, s), jnp.float32),
        in_specs=[
            pl.BlockSpec(memory_space=pltpu.VMEM),
            pl.BlockSpec(memory_space=pltpu.VMEM),
            pl.BlockSpec(memory_space=pltpu.VMEM),
        ],
        out_specs=pl.BlockSpec(memory_space=pltpu.VMEM),
        scratch_shapes=[
            pltpu.VMEM((N_DEV, b, h, d, s), jnp.float32),
            pltpu.VMEM((N_DEV, b, h, d, s), jnp.float32),
            pltpu.SemaphoreType.DMA((N_DEV - 1,)),
            pltpu.SemaphoreType.DMA((N_DEV - 1,)),
            pltpu.SemaphoreType.DMA((N_DEV - 1,)),
            pltpu.SemaphoreType.DMA((N_DEV - 1,)),
        ],
        compiler_params=pltpu.CompilerParams(collective_id=0),
    )(qT, kT, vT)

    return jnp.transpose(oT, (0, 3, 1, 2))


# device time: 185806 ns/iter; 1.0048x vs baseline; 1.0048x over previous
import jax
import jax.numpy as jnp
from jax import lax
from jax.experimental import pallas as pl
from jax.experimental.pallas import tpu as pltpu

N_DEV = 4


def kernel(Q, K, V):
    b, s, h, d = Q.shape
    scale = d ** -0.5

    def body(qT_ref, kT_ref, vT_ref, oT_ref, kbuf, vbuf,
             ksend, krecv, vsend, vrecv):
        my_x = lax.axis_index("x")
        my_y = lax.axis_index("y")
        my_z = lax.axis_index("z")
        left = (my_z - 1) % N_DEV
        right = (my_z + 1) % N_DEV

        barrier = pltpu.get_barrier_semaphore()
        for nbr in (left, right):
            pl.semaphore_signal(
                barrier, inc=1,
                device_id=(my_x, my_y, nbr),
                device_id_type=pl.DeviceIdType.MESH,
            )
        pl.semaphore_wait(barrier, 2)

        kbuf[0] = kT_ref[...]
        vbuf[0] = vT_ref[...]

        for hop in range(N_DEV - 1):
            k_rdma = pltpu.make_async_remote_copy(
                src_ref=kbuf.at[hop],
                dst_ref=kbuf.at[hop + 1],
                send_sem=ksend.at[hop],
                recv_sem=krecv.at[hop],
                device_id=(my_x, my_y, right),
                device_id_type=pl.DeviceIdType.MESH,
            )
            v_rdma = pltpu.make_async_remote_copy(
                src_ref=vbuf.at[hop],
                dst_ref=vbuf.at[hop + 1],
                send_sem=vsend.at[hop],
                recv_sem=vrecv.at[hop],
                device_id=(my_x, my_y, left),
                device_id_type=pl.DeviceIdType.MESH,
            )
            k_rdma.start()
            v_rdma.start()
            k_rdma.wait()
            v_rdma.wait()

        def loop_body(i, carry):
            bb = i // h
            hh = i % h
            qT = qT_ref[bb, hh]
            kT = jnp.concatenate(
                [kbuf[sl, bb, hh] for sl in range(N_DEV)], axis=-1
            )
            vT = jnp.concatenate(
                [vbuf[(N_DEV - sl) % N_DEV, bb, hh] for sl in range(N_DEV)],
                axis=-1,
            )
            sc = lax.dot_general(
                qT, kT, (((0,), (0,)), ((), ())),
                preferred_element_type=jnp.float32,
            ) * scale
            m = jnp.max(sc, axis=1, keepdims=True)
            p = jnp.exp(sc - m)
            p = p / jnp.sum(p, axis=1, keepdims=True)
            oT = lax.dot_general(
                vT, p, (((1,), (1,)), ((), ())),
                preferred_element_type=jnp.float32,
            )
            oT_ref[bb, hh] = oT
            return carry

        lax.fori_loop(0, b * h, loop_body, 0)

    qT = jnp.transpose(Q, (0, 2, 3, 1))
    kT = jnp.transpose(K, (0, 2, 3, 1))
    vT = jnp.transpose(V, (0, 2, 3, 1))

    oT = pl.pallas_call(
        body,
        out_shape=jax.ShapeDtypeStruct((b, h, d, s), jnp.float32),
        in_specs=[
            pl.BlockSpec(memory_space=pltpu.VMEM),
            pl.BlockSpec(memory_space=pltpu.VMEM),
            pl.BlockSpec(memory_space=pltpu.VMEM),
        ],
        out_specs=pl.BlockSpec(memory_space=pltpu.VMEM),
        scratch_shapes=[
            pltpu.VMEM((N_DEV, b, h, d, s), jnp.float32),
            pltpu.VMEM((N_DEV, b, h, d, s), jnp.float32),
            pltpu.SemaphoreType.DMA((N_DEV - 1,)),
            pltpu.SemaphoreType.DMA((N_DEV - 1,)),
            pltpu.SemaphoreType.DMA((N_DEV - 1,)),
            pltpu.SemaphoreType.DMA((N_DEV - 1,)),
        ],
        compiler_params=pltpu.CompilerParams(collective_id=0),
    )(qT, kT, vT)

    return jnp.transpose(oT, (0, 3, 1, 2))


# device time: 105294 ns/iter; 1.7730x vs baseline; 1.7646x over previous
import os

import jax
import jax.numpy as jnp
from jax import lax
from jax.experimental import pallas as pl
from jax.experimental.pallas import tpu as pltpu

Z = 4
_COMM_ONLY = os.environ.get("COMM_ONLY") == "1"


def kernel(Q, K, V):
    b, s, h, d = Q.shape
    scale = d ** -0.5

    def body(qT_ref, kT_ref, vT_ref, oT_ref, kbuf, vbuf, l_buf, acc_buf,
             zr_send, zr_recv, zl_send, zl_recv, x_send, x_recv):
        my_x = lax.axis_index("x")
        my_y = lax.axis_index("y")
        my_z = lax.axis_index("z")
        has_left = my_z > 0
        has_right = my_z < Z - 1
        left_z = jnp.maximum(my_z - 1, 0)
        right_z = jnp.minimum(my_z + 1, Z - 1)
        partner_x = 1 - my_x

        barrier = pltpu.get_barrier_semaphore()
        pl.semaphore_signal(
            barrier, inc=1,
            device_id=(partner_x, my_y, my_z),
            device_id_type=pl.DeviceIdType.MESH,
        )

        @pl.when(has_left)
        def _():
            pl.semaphore_signal(
                barrier, inc=1,
                device_id=(my_x, my_y, left_z),
                device_id_type=pl.DeviceIdType.MESH,
            )

        @pl.when(has_right)
        def _():
            pl.semaphore_signal(
                barrier, inc=1,
                device_id=(my_x, my_y, right_z),
                device_id_type=pl.DeviceIdType.MESH,
            )

        n_peers = 1 + has_left.astype(jnp.int32) + has_right.astype(jnp.int32)
        pl.semaphore_wait(barrier, n_peers)

        kbuf[my_z] = kT_ref[...]
        vbuf[my_z] = vT_ref[...]

        def x_forward(gbuf, origin, hf):
            fwd = pltpu.make_async_remote_copy(
                src_ref=gbuf.at[origin, hf],
                dst_ref=gbuf.at[origin, hf],
                send_sem=x_send.at[origin, hf],
                recv_sem=x_recv.at[origin, hf],
                device_id=(partner_x, my_y, my_z),
                device_id_type=pl.DeviceIdType.MESH,
            )
            fwd.start()

        def z_phase(gbuf, mid_compute):

            def sends(st, hf):
                r_origin = jnp.maximum(my_z - st, 0)
                send_r = has_right & (my_z - st >= 0)
                l_origin = jnp.minimum(my_z + st, Z - 1)
                send_l = has_left & (my_z + st <= Z - 1)

                r_rdma = pltpu.make_async_remote_copy(
                    src_ref=gbuf.at[r_origin, hf],
                    dst_ref=gbuf.at[r_origin, hf],
                    send_sem=zr_send.at[st, hf],
                    recv_sem=zr_recv.at[st, hf],
                    device_id=(my_x, my_y, right_z),
                    device_id_type=pl.DeviceIdType.MESH,
                )
                l_rdma = pltpu.make_async_remote_copy(
                    src_ref=gbuf.at[l_origin, hf],
                    dst_ref=gbuf.at[l_origin, hf],
                    send_sem=zl_send.at[st, hf],
                    recv_sem=zl_recv.at[st, hf],
                    device_id=(my_x, my_y, left_z),
                    device_id_type=pl.DeviceIdType.MESH,
                )

                @pl.when(send_r)
                def _():
                    r_rdma.start()

                @pl.when(send_l)
                def _():
                    l_rdma.start()

            def recv_and_fwd(st, hf):
                fl_origin = jnp.maximum(my_z - 1 - st, 0)
                rcv_l = has_left & (my_z - 1 - st >= 0)
                from_l = pltpu.make_async_remote_copy(
                    src_ref=gbuf.at[fl_origin, hf],
                    dst_ref=gbuf.at[fl_origin, hf],
                    send_sem=zr_send.at[st, hf],
                    recv_sem=zr_recv.at[st, hf],
                    device_id=(my_x, my_y, left_z),
                    device_id_type=pl.DeviceIdType.MESH,
                )
                fr_origin = jnp.minimum(my_z + 1 + st, Z - 1)
                rcv_r = has_right & (my_z + 1 + st <= Z - 1)
                from_r = pltpu.make_async_remote_copy(
                    src_ref=gbuf.at[fr_origin, hf],
                    dst_ref=gbuf.at[fr_origin, hf],
                    send_sem=zl_send.at[st, hf],
                    recv_sem=zl_recv.at[st, hf],
                    device_id=(my_x, my_y, right_z),
                    device_id_type=pl.DeviceIdType.MESH,
                )

                @pl.when(rcv_l)
                def _():
                    from_l.wait_recv()
                    x_forward(gbuf, fl_origin, hf)

                @pl.when(rcv_r)
                def _():
                    from_r.wait_recv()
                    x_forward(gbuf, fr_origin, hf)

            for hf in range(b):
                sends(0, hf)
            for st in range(1, Z - 1):
                for hf in range(b):
                    recv_and_fwd(st - 1, hf)
                    sends(st, hf)
                if st == 1:
                    mid_compute()
            for hf in range(b):
                recv_and_fwd(Z - 2, hf)

            for st in range(Z - 1):
                for hf in range(b):
                    r_rdma = pltpu.make_async_remote_copy(
                        src_ref=gbuf.at[0, hf], dst_ref=gbuf.at[0, hf],
                        send_sem=zr_send.at[st, hf],
                        recv_sem=zr_recv.at[st, hf],
                        device_id=(my_x, my_y, right_z),
                        device_id_type=pl.DeviceIdType.MESH,
                    )
                    l_rdma = pltpu.make_async_remote_copy(
                        src_ref=gbuf.at[0, hf], dst_ref=gbuf.at[0, hf],
                        send_sem=zl_send.at[st, hf],
                        recv_sem=zl_recv.at[st, hf],
                        device_id=(my_x, my_y, left_z),
                        device_id_type=pl.DeviceIdType.MESH,
                    )

                    @pl.when(has_right & (my_z - st >= 0))
                    def _():
                        r_rdma.wait_send()

                    @pl.when(has_left & (my_z + st <= Z - 1))
                    def _():
                        l_rdma.wait_send()

        def x_send_drain(gbuf):
            for o in range(Z):
                for hf in range(b):
                    snd = pltpu.make_async_remote_copy(
                        src_ref=gbuf.at[o, hf], dst_ref=gbuf.at[o, hf],
                        send_sem=x_send.at[o, hf], recv_sem=x_recv.at[o, hf],
                        device_id=(partner_x, my_y, my_z),
                        device_id_type=pl.DeviceIdType.MESH,
                    )

                    @pl.when(o != my_z)
                    def _():
                        snd.wait_send()

        def wait_x_chunk(o):
            for hf in range(b):
                rcv = pltpu.make_async_remote_copy(
                    src_ref=vbuf.at[o, hf], dst_ref=vbuf.at[o, hf],
                    send_sem=x_send.at[o, hf], recv_sem=x_recv.at[o, hf],
                    device_id=(partner_x, my_y, my_z),
                    device_id_type=pl.DeviceIdType.MESH,
                )
                rcv.wait_recv()

        def flash_chunk(o, first, last):
            def inner(i, carry):
                bb = i // h
                hh = i % h
                qT = qT_ref[bb, hh].astype(jnp.bfloat16)
                scT = lax.dot_general(
                    kbuf[o, bb, hh].astype(jnp.bfloat16), qT,
                    (((0,), (0,)), ((), ())),
                    preferred_element_type=jnp.float32,
                ) * scale
                p = jnp.exp(scT)
                pl_sum = jnp.sum(p, axis=0, keepdims=True)
                pv = lax.dot_general(
                    vbuf[o, bb, hh].astype(jnp.bfloat16),
                    p.astype(jnp.bfloat16),
                    (((1,), (0,)), ((), ())),
                    preferred_element_type=jnp.float32,
                )
                if first:
                    l_new = pl_sum
                    acc_new = pv
                else:
                    l_new = l_buf[i] + pl_sum
                    acc_new = acc_buf[bb, hh] + pv
                if last:
                    oT_ref[bb, hh] = acc_new / l_new
                else:
                    l_buf[i] = l_new
                    acc_buf[bb, hh] = acc_new
                return carry

            lax.fori_loop(0, b * h, inner, 0)

        def own_flash():
            for k in range(Z):
                @pl.when(my_z == k)
                def _(k=k):
                    flash_chunk(k, first=True, last=False)

        mid_compute = (lambda: None) if _COMM_ONLY else own_flash

        @pl.when(my_x == 0)
        def _():
            z_phase(kbuf, mid_compute)

        @pl.when(my_x == 1)
        def _():
            z_phase(vbuf, mid_compute)

        if _COMM_ONLY:
            for o in range(Z):
                @pl.when(o != my_z)
                def _(o=o):
                    wait_x_chunk(o)
            oT_ref[...] = qT_ref[...] + kbuf[0] + vbuf[0]
        else:
            _ORDER = {0: [1, 2, 3], 1: [0, 2, 3], 2: [1, 3, 0], 3: [2, 1, 0]}
            for k in range(Z):
                @pl.when(my_z == k)
                def _(k=k):
                    order = _ORDER[k]
                    for j, o in enumerate(order):
                        wait_x_chunk(o)
                        flash_chunk(o, first=False, last=(j == len(order) - 1))

        @pl.when(my_x == 0)
        def _():
            x_send_drain(kbuf)

        @pl.when(my_x == 1)
        def _():
            x_send_drain(vbuf)

    qT = jnp.transpose(Q, (0, 2, 3, 1))
    kT = jnp.transpose(K, (0, 2, 3, 1))
    vT = jnp.transpose(V, (0, 2, 3, 1))

    oT = pl.pallas_call(
        body,
        out_shape=jax.ShapeDtypeStruct((b, h, d, s), jnp.float32),
        in_specs=[
            pl.BlockSpec(memory_space=pltpu.VMEM),
            pl.BlockSpec(memory_space=pltpu.VMEM),
            pl.BlockSpec(memory_space=pltpu.VMEM),
        ],
        out_specs=pl.BlockSpec(memory_space=pltpu.VMEM),
        scratch_shapes=[
            pltpu.VMEM((Z, b, h, d, s), jnp.float32),
            pltpu.VMEM((Z, b, h, d, s), jnp.float32),
            pltpu.VMEM((b * h, 1, s), jnp.float32),
            pltpu.VMEM((b, h, d, s), jnp.float32),
            pltpu.SemaphoreType.DMA((Z - 1, b)),
            pltpu.SemaphoreType.DMA((Z - 1, b)),
            pltpu.SemaphoreType.DMA((Z - 1, b)),
            pltpu.SemaphoreType.DMA((Z - 1, b)),
            pltpu.SemaphoreType.DMA((Z, b)),
            pltpu.SemaphoreType.DMA((Z, b)),
        ],
        compiler_params=pltpu.CompilerParams(collective_id=0),
    )(qT, kT, vT)

    return jnp.transpose(oT, (0, 3, 1, 2))


# device time: 105030 ns/iter; 1.7775x vs baseline; 1.0025x over previous
import os

import jax
import jax.numpy as jnp
from jax import lax
from jax.experimental import pallas as pl
from jax.experimental.pallas import tpu as pltpu

Z = 4
_COMM_ONLY = os.environ.get("COMM_ONLY") == "1"


def kernel(Q, K, V):
    b, s, h, d = Q.shape
    scale = d ** -0.5

    def body(qT_ref, kT_ref, vT_ref, oT_ref, kbuf, vbuf, l_buf, acc_buf,
             zr_send, zr_recv, zl_send, zl_recv, x_send, x_recv):
        my_x = lax.axis_index("x")
        my_y = lax.axis_index("y")
        my_z = lax.axis_index("z")
        has_left = my_z > 0
        has_right = my_z < Z - 1
        left_z = jnp.maximum(my_z - 1, 0)
        right_z = jnp.minimum(my_z + 1, Z - 1)
        partner_x = 1 - my_x

        barrier = pltpu.get_barrier_semaphore()
        pl.semaphore_signal(
            barrier, inc=1,
            device_id=(partner_x, my_y, my_z),
            device_id_type=pl.DeviceIdType.MESH,
        )

        @pl.when(has_left)
        def _():
            pl.semaphore_signal(
                barrier, inc=1,
                device_id=(my_x, my_y, left_z),
                device_id_type=pl.DeviceIdType.MESH,
            )

        @pl.when(has_right)
        def _():
            pl.semaphore_signal(
                barrier, inc=1,
                device_id=(my_x, my_y, right_z),
                device_id_type=pl.DeviceIdType.MESH,
            )

        n_peers = 1 + has_left.astype(jnp.int32) + has_right.astype(jnp.int32)
        pl.semaphore_wait(barrier, n_peers)

        def x_forward(gbuf, origin, hf):
            fwd = pltpu.make_async_remote_copy(
                src_ref=gbuf.at[origin, hf],
                dst_ref=gbuf.at[origin, hf],
                send_sem=x_send.at[origin, hf],
                recv_sem=x_recv.at[origin, hf],
                device_id=(partner_x, my_y, my_z),
                device_id_type=pl.DeviceIdType.MESH,
            )
            fwd.start()

        def z_phase(gbuf, src0, mid_compute):

            def sends(st, hf):
                r_origin = jnp.maximum(my_z - st, 0)
                send_r = has_right & (my_z - st >= 0)
                l_origin = jnp.minimum(my_z + st, Z - 1)
                send_l = has_left & (my_z + st <= Z - 1)

                r_src = src0.at[hf] if st == 0 else gbuf.at[r_origin, hf]
                l_src = src0.at[hf] if st == 0 else gbuf.at[l_origin, hf]
                r_rdma = pltpu.make_async_remote_copy(
                    src_ref=r_src,
                    dst_ref=gbuf.at[r_origin, hf],
                    send_sem=zr_send.at[st, hf],
                    recv_sem=zr_recv.at[st, hf],
                    device_id=(my_x, my_y, right_z),
                    device_id_type=pl.DeviceIdType.MESH,
                )
                l_rdma = pltpu.make_async_remote_copy(
                    src_ref=l_src,
                    dst_ref=gbuf.at[l_origin, hf],
                    send_sem=zl_send.at[st, hf],
                    recv_sem=zl_recv.at[st, hf],
                    device_id=(my_x, my_y, left_z),
                    device_id_type=pl.DeviceIdType.MESH,
                )

                @pl.when(send_r)
                def _():
                    r_rdma.start()

                @pl.when(send_l)
                def _():
                    l_rdma.start()

            def recv_and_fwd(st, hf):
                fl_origin = jnp.maximum(my_z - 1 - st, 0)
                rcv_l = has_left & (my_z - 1 - st >= 0)
                from_l = pltpu.make_async_remote_copy(
                    src_ref=gbuf.at[fl_origin, hf],
                    dst_ref=gbuf.at[fl_origin, hf],
                    send_sem=zr_send.at[st, hf],
                    recv_sem=zr_recv.at[st, hf],
                    device_id=(my_x, my_y, left_z),
                    device_id_type=pl.DeviceIdType.MESH,
                )
                fr_origin = jnp.minimum(my_z + 1 + st, Z - 1)
                rcv_r = has_right & (my_z + 1 + st <= Z - 1)
                from_r = pltpu.make_async_remote_copy(
                    src_ref=gbuf.at[fr_origin, hf],
                    dst_ref=gbuf.at[fr_origin, hf],
                    send_sem=zl_send.at[st, hf],
                    recv_sem=zl_recv.at[st, hf],
                    device_id=(my_x, my_y, right_z),
                    device_id_type=pl.DeviceIdType.MESH,
                )

                @pl.when(rcv_l)
                def _():
                    from_l.wait_recv()
                    x_forward(gbuf, fl_origin, hf)

                @pl.when(rcv_r)
                def _():
                    from_r.wait_recv()
                    x_forward(gbuf, fr_origin, hf)

            for hf in range(b):
                sends(0, hf)
            for st in range(1, Z - 1):
                for hf in range(b):
                    recv_and_fwd(st - 1, hf)
                    sends(st, hf)
                if st == 1:
                    mid_compute()
            for hf in range(b):
                recv_and_fwd(Z - 2, hf)

            for st in range(Z - 1):
                for hf in range(b):
                    r_rdma = pltpu.make_async_remote_copy(
                        src_ref=gbuf.at[0, hf], dst_ref=gbuf.at[0, hf],
                        send_sem=zr_send.at[st, hf],
                        recv_sem=zr_recv.at[st, hf],
                        device_id=(my_x, my_y, right_z),
                        device_id_type=pl.DeviceIdType.MESH,
                    )
                    l_rdma = pltpu.make_async_remote_copy(
                        src_ref=gbuf.at[0, hf], dst_ref=gbuf.at[0, hf],
                        send_sem=zl_send.at[st, hf],
                        recv_sem=zl_recv.at[st, hf],
                        device_id=(my_x, my_y, left_z),
                        device_id_type=pl.DeviceIdType.MESH,
                    )

                    @pl.when(has_right & (my_z - st >= 0))
                    def _():
                        r_rdma.wait_send()

                    @pl.when(has_left & (my_z + st <= Z - 1))
                    def _():
                        l_rdma.wait_send()

        def x_send_drain(gbuf):
            for o in range(Z):
                for hf in range(b):
                    snd = pltpu.make_async_remote_copy(
                        src_ref=gbuf.at[o, hf], dst_ref=gbuf.at[o, hf],
                        send_sem=x_send.at[o, hf], recv_sem=x_recv.at[o, hf],
                        device_id=(partner_x, my_y, my_z),
                        device_id_type=pl.DeviceIdType.MESH,
                    )

                    @pl.when(o != my_z)
                    def _():
                        snd.wait_send()

        def wait_x_chunk(o):
            for hf in range(b):
                rcv = pltpu.make_async_remote_copy(
                    src_ref=vbuf.at[o, hf], dst_ref=vbuf.at[o, hf],
                    send_sem=x_send.at[o, hf], recv_sem=x_recv.at[o, hf],
                    device_id=(partner_x, my_y, my_z),
                    device_id_type=pl.DeviceIdType.MESH,
                )
                rcv.wait_recv()

        def flash_chunk(k_at, v_at, first, last):
            def inner(i, carry):
                bb = i // h
                hh = i % h
                qT = qT_ref[bb, hh].astype(jnp.bfloat16)
                scT = lax.dot_general(
                    k_at(bb, hh).astype(jnp.bfloat16), qT,
                    (((0,), (0,)), ((), ())),
                    preferred_element_type=jnp.float32,
                ) * scale
                p = jnp.exp(scT)
                pl_sum = jnp.sum(p, axis=0, keepdims=True)
                pv = lax.dot_general(
                    v_at(bb, hh).astype(jnp.bfloat16),
                    p.astype(jnp.bfloat16),
                    (((1,), (0,)), ((), ())),
                    preferred_element_type=jnp.float32,
                )
                if first:
                    l_new = pl_sum
                    acc_new = pv
                else:
                    l_new = l_buf[i] + pl_sum
                    acc_new = acc_buf[bb, hh] + pv
                if last:
                    oT_ref[bb, hh] = acc_new / l_new
                else:
                    l_buf[i] = l_new
                    acc_buf[bb, hh] = acc_new
                return carry

            lax.fori_loop(0, b * h, inner, 0)

        def own_flash():
            flash_chunk(
                lambda bb, hh: kT_ref[bb, hh],
                lambda bb, hh: vT_ref[bb, hh],
                first=True, last=False,
            )

        mid_compute = (lambda: None) if _COMM_ONLY else own_flash

        @pl.when(my_x == 0)
        def _():
            z_phase(kbuf, kT_ref, mid_compute)

        @pl.when(my_x == 1)
        def _():
            z_phase(vbuf, vT_ref, mid_compute)

        if _COMM_ONLY:
            for o in range(Z):
                @pl.when(o != my_z)
                def _(o=o):
                    wait_x_chunk(o)
            oT_ref[...] = qT_ref[...] + kbuf[0] + vbuf[0]
        else:
            _ORDER = {0: [1, 2, 3], 1: [0, 2, 3], 2: [1, 3, 0], 3: [2, 1, 0]}
            for k in range(Z):
                @pl.when(my_z == k)
                def _(k=k):
                    order = _ORDER[k]
                    for j, o in enumerate(order):
                        wait_x_chunk(o)
                        flash_chunk(
                            lambda bb, hh, o=o: kbuf[o, bb, hh],
                            lambda bb, hh, o=o: vbuf[o, bb, hh],
                            first=False, last=(j == len(order) - 1),
                        )

        @pl.when(my_x == 0)
        def _():
            x_send_drain(kbuf)

        @pl.when(my_x == 1)
        def _():
            x_send_drain(vbuf)

    qT = jnp.transpose(Q, (0, 2, 3, 1))
    kT = jnp.transpose(K, (0, 2, 3, 1))
    vT = jnp.transpose(V, (0, 2, 3, 1))

    oT = pl.pallas_call(
        body,
        out_shape=jax.ShapeDtypeStruct((b, h, d, s), jnp.float32),
        in_specs=[
            pl.BlockSpec(memory_space=pltpu.VMEM),
            pl.BlockSpec(memory_space=pltpu.VMEM),
            pl.BlockSpec(memory_space=pltpu.VMEM),
        ],
        out_specs=pl.BlockSpec(memory_space=pltpu.VMEM),
        scratch_shapes=[
            pltpu.VMEM((Z, b, h, d, s), jnp.float32),
            pltpu.VMEM((Z, b, h, d, s), jnp.float32),
            pltpu.VMEM((b * h, 1, s), jnp.float32),
            pltpu.VMEM((b, h, d, s), jnp.float32),
            pltpu.SemaphoreType.DMA((Z - 1, b)),
            pltpu.SemaphoreType.DMA((Z - 1, b)),
            pltpu.SemaphoreType.DMA((Z - 1, b)),
            pltpu.SemaphoreType.DMA((Z - 1, b)),
            pltpu.SemaphoreType.DMA((Z, b)),
            pltpu.SemaphoreType.DMA((Z, b)),
        ],
        compiler_params=pltpu.CompilerParams(collective_id=0),
    )(qT, kT, vT)

    return jnp.transpose(oT, (0, 3, 1, 2))
